# baseline (speedup 1.0000x reference)
import functools

import jax
import jax.numpy as jnp
from jax import lax
from jax.experimental import pallas as pl
from jax.experimental.pallas import tpu as pltpu
from jax.experimental.pallas import tpu_sc as plsc

SUBVERT_NUM = 100000
EMB_DIM = 64
NUM_FILTERS = 128
BATCH = 16384

_info = plsc.get_sparse_core_info()
_NC, _NS = _info.num_cores, _info.num_subcores
_NW = _NC * _NS
_B_PER_W = BATCH // _NW

_T_BLOCK = 2048
_T_GRID = 25
_D_ROWS = _T_BLOCK * _T_GRID


def _untranspose_body(lo_ref, hi_ref, o_ref):
    ii = lax.broadcasted_iota(jnp.int32, (EMB_DIM, EMB_DIM), 0)
    jj = lax.broadcasted_iota(jnp.int32, (EMB_DIM, EMB_DIM), 1)
    eye = jnp.where(ii == jj, 1.0, 0.0).astype(jnp.float32)
    o_ref[:, 0:EMB_DIM] = lax.dot_general(
        lo_ref[...], eye,
        dimension_numbers=(((0,), (0,)), ((), ())),
        preferred_element_type=jnp.float32,
    )
    o_ref[:, EMB_DIM:2 * EMB_DIM] = lax.dot_general(
        hi_ref[...], eye,
        dimension_numbers=(((0,), (0,)), ((), ())),
        preferred_element_type=jnp.float32,
    )


def _untranspose_tc(tableT):
    return pl.pallas_call(
        _untranspose_body,
        grid=(_T_GRID,),
        in_specs=[
            pl.BlockSpec((EMB_DIM, _T_BLOCK), lambda i: (0, i)),
            pl.BlockSpec(
                (EMB_DIM, _T_BLOCK),
                lambda i: (0, jnp.minimum(i + _T_GRID,
                                          SUBVERT_NUM // _T_BLOCK)),
            ),
        ],
        out_specs=pl.BlockSpec((_T_BLOCK, 2 * EMB_DIM), lambda i: (i, 0)),
        out_shape=jax.ShapeDtypeStruct((_D_ROWS, 2 * EMB_DIM), jnp.float32),
    )(tableT, tableT)


def _gather_sc(idx2, dense64):
    @functools.partial(
        pl.kernel,
        mesh=plsc.VectorSubcoreMesh(core_axis_name="c", subcore_axis_name="s"),
        out_type=jax.ShapeDtypeStruct((BATCH, EMB_DIM), jnp.float32),
        scratch_types=[
            pltpu.VMEM((_B_PER_W,), jnp.int32),
            pltpu.VMEM((_B_PER_W, EMB_DIM), jnp.float32),
            pltpu.SemaphoreType.DMA,
            pltpu.SemaphoreType.DMA,
        ],
        compiler_params=pltpu.CompilerParams(use_tc_tiling_on_sc=False),
    )
    def k(idx_hbm, dense_hbm, out_hbm, idx_v, rows_v, sem_i, sem):
        wid = lax.axis_index("s") * _NC + lax.axis_index("c")
        base = wid * _B_PER_W
        pltpu.async_copy(idx_hbm.at[wid], idx_v, sem_i).wait()

        def body(j, _):
            vbase = j * 16
            idx_vec = idx_v[pl.ds(vbase, 16)]
            row_vec = jnp.where(
                idx_vec < _D_ROWS,
                idx_vec * 2,
                (idx_vec - _D_ROWS) * 2 + 1,
            )
            for t in range(16):
                r = row_vec[t]
                pltpu.async_copy(
                    dense_hbm.at[pl.ds(r, 1)], rows_v.at[pl.ds(vbase + t, 1)], sem
                )
            return 0

        lax.fori_loop(0, _B_PER_W // 16, body, 0)

        pltpu.make_async_copy(
            dense_hbm.at[pl.ds(0, _B_PER_W)], rows_v, sem
        ).wait()
        pltpu.sync_copy(rows_v, out_hbm.at[pl.ds(base, _B_PER_W)])

    return k(idx2, dense64)


def _proj_body(x_ref, wt_ref, b_ref, o_ref):
    o_ref[...] = (
        lax.dot_general(
            x_ref[...], wt_ref[...],
            dimension_numbers=(((1,), (0,)), ((), ())),
            preferred_element_type=jnp.float32,
        )
        + b_ref[...]
    )


_TC_BLOCK = 4096


def _project_tc(x, Wt, b2):
    grid = BATCH // _TC_BLOCK
    return pl.pallas_call(
        _proj_body,
        grid=(grid,),
        in_specs=[
            pl.BlockSpec((_TC_BLOCK, EMB_DIM), lambda i: (i, 0)),
            pl.BlockSpec((EMB_DIM, NUM_FILTERS), lambda i: (0, 0)),
            pl.BlockSpec((1, NUM_FILTERS), lambda i: (0, 0)),
        ],
        out_specs=pl.BlockSpec((_TC_BLOCK, NUM_FILTERS), lambda i: (i, 0)),
        out_shape=jax.ShapeDtypeStruct((BATCH, NUM_FILTERS), jnp.float32),
    )(x, Wt, b2)


def kernel(input_subvert, table, W, b):
    idx2 = input_subvert.astype(jnp.int32).reshape(_NW, _B_PER_W)
    dense64 = _untranspose_tc(table.T).reshape(2 * _D_ROWS, EMB_DIM)
    gathered = _gather_sc(idx2, dense64)
    return _project_tc(gathered, W.T, b.reshape(1, NUM_FILTERS))


# speedup vs baseline: 1.0784x; 1.0784x over previous
import functools

import jax
import jax.numpy as jnp
from jax import lax
from jax.experimental import pallas as pl
from jax.experimental.pallas import tpu as pltpu
from jax.experimental.pallas import tpu_sc as plsc

SUBVERT_NUM = 100000
EMB_DIM = 64
NUM_FILTERS = 128
BATCH = 16384

_info = plsc.get_sparse_core_info()
_NC, _NS = _info.num_cores, _info.num_subcores
_NW = _NC * _NS
_B_PER_W = BATCH // _NW

_T_BLOCK = 2048
_T_GRID = 25
_D_ROWS = _T_BLOCK * _T_GRID


def _untranspose_body(lo_ref, hi_ref, o_ref):
    o_ref[:, 0:EMB_DIM] = lo_ref[...].T
    o_ref[:, EMB_DIM:2 * EMB_DIM] = hi_ref[...].T


def _untranspose_tc(tableT):
    return pl.pallas_call(
        _untranspose_body,
        grid=(_T_GRID,),
        in_specs=[
            pl.BlockSpec((EMB_DIM, _T_BLOCK), lambda i: (0, i)),
            pl.BlockSpec(
                (EMB_DIM, _T_BLOCK),
                lambda i: (0, jnp.minimum(i + _T_GRID,
                                          SUBVERT_NUM // _T_BLOCK)),
            ),
        ],
        out_specs=pl.BlockSpec((_T_BLOCK, 2 * EMB_DIM), lambda i: (i, 0)),
        out_shape=jax.ShapeDtypeStruct((_D_ROWS, 2 * EMB_DIM), jnp.float32),
        compiler_params=pltpu.CompilerParams(fuse_transposed_lhs_in_matmul=True),
    )(tableT, tableT)


def _gather_sc(idx2, dense64):
    @functools.partial(
        pl.kernel,
        mesh=plsc.VectorSubcoreMesh(core_axis_name="c", subcore_axis_name="s"),
        out_type=jax.ShapeDtypeStruct((BATCH, 2 * EMB_DIM), jnp.float32),
        scratch_types=[
            pltpu.VMEM((_B_PER_W,), jnp.int32),
            pltpu.VMEM((_B_PER_W, EMB_DIM), jnp.float32),
            pltpu.SemaphoreType.DMA,
            pltpu.SemaphoreType.DMA,
        ],
        compiler_params=pltpu.CompilerParams(use_tc_tiling_on_sc=False),
    )
    def k(idx_hbm, dense_hbm, out_hbm, idx_v, rows_v, sem_i, sem):
        wid = lax.axis_index("s") * _NC + lax.axis_index("c")
        base = wid * _B_PER_W
        pltpu.async_copy(idx_hbm.at[wid], idx_v, sem_i).wait()

        def body(j, _):
            vbase = j * 16
            idx_vec = idx_v[pl.ds(vbase, 16)]
            row_vec = jnp.where(
                idx_vec < _D_ROWS,
                idx_vec * 2,
                (idx_vec - _D_ROWS) * 2 + 1,
            )
            for t in range(16):
                r = row_vec[t]
                pltpu.async_copy(
                    dense_hbm.at[pl.ds(r, 1)], rows_v.at[pl.ds(vbase + t, 1)], sem
                )
            return 0

        lax.fori_loop(0, _B_PER_W // 16, body, 0)

        pltpu.make_async_copy(
            dense_hbm.at[pl.ds(0, _B_PER_W)], rows_v, sem
        ).wait()
        pltpu.sync_copy(
            rows_v, out_hbm.at[pl.ds(base, _B_PER_W), pl.ds(0, EMB_DIM)]
        )

    return k(idx2, dense64)


def _proj_body(x_ref, wt_ref, b_ref, o_ref):
    o_ref[...] = (
        lax.dot_general(
            x_ref[:, 0:EMB_DIM], wt_ref[...],
            dimension_numbers=(((1,), (0,)), ((), ())),
            preferred_element_type=jnp.float32,
        )
        + b_ref[...]
    )


_TC_BLOCK = 2048


def _project_tc(x, Wt, b2):
    grid = BATCH // _TC_BLOCK
    return pl.pallas_call(
        _proj_body,
        grid=(grid,),
        in_specs=[
            pl.BlockSpec((_TC_BLOCK, 2 * EMB_DIM), lambda i: (i, 0)),
            pl.BlockSpec((EMB_DIM, NUM_FILTERS), lambda i: (0, 0)),
            pl.BlockSpec((1, NUM_FILTERS), lambda i: (0, 0)),
        ],
        out_specs=pl.BlockSpec((_TC_BLOCK, NUM_FILTERS), lambda i: (i, 0)),
        out_shape=jax.ShapeDtypeStruct((BATCH, NUM_FILTERS), jnp.float32),
    )(x, Wt, b2)


def kernel(input_subvert, table, W, b):
    idx2 = input_subvert.astype(jnp.int32).reshape(_NW, _B_PER_W)
    dense64 = _untranspose_tc(table.T).reshape(2 * _D_ROWS, EMB_DIM)
    gathered = _gather_sc(idx2, dense64)
    return _project_tc(gathered, W.T, b.reshape(1, NUM_FILTERS))
